# baseline (speedup 1.0000x reference)
import jax
import jax.numpy as jnp
from jax.experimental import pallas as pl


def _add_kernel(x_ref, a_ref, b_ref, o_ref):
    BLK, D = x_ref.shape
    half = D // 2
    sa = a_ref[0:1, :half]
    ca = a_ref[0:1, half:]
    sb = b_ref[:, :half]
    cb = b_ref[:, half:]
    e_sin = sa * cb + ca * sb
    e_cos = ca * cb - sa * sb
    e = jnp.concatenate([e_sin, e_cos], axis=-1)
    o_ref[...] = x_ref[...] + e


def kernel(x, embed):
    B, S, D = x.shape
    xf = x.reshape(B * S, D)
    BLK = 512
    SB = S // BLK
    grid = (B * SB,)
    out = pl.pallas_call(
        _add_kernel,
        grid=grid,
        in_specs=[
            pl.BlockSpec((BLK, D), lambda i: (i, 0)),
            pl.BlockSpec((8, D), lambda i: (BLK // 8 * (i % SB), 0)),
            pl.BlockSpec((BLK, D), lambda i: (0, 0)),
        ],
        out_specs=pl.BlockSpec((BLK, D), lambda i: (i, 0)),
        out_shape=jax.ShapeDtypeStruct(xf.shape, xf.dtype),
    )(xf, embed, embed)
    return out.reshape(B, S, D)
